# baseline (speedup 1.0000x reference)
import functools

import jax
import jax.numpy as jnp
from jax import lax
from jax.experimental import pallas as pl
from jax.experimental.pallas import tpu as pltpu
from jax.experimental.pallas import tpu_sc as plsc

BATCH = 4096
SEQ = 200
DIM = 64
NC = 2
NS = 16
NW = NC * NS
SEQ_PER_W = BATCH // NW
IDX_PER_W = SEQ_PER_W * SEQ


def _sc_embed(x_flat, token_table, pos_table):
    mesh = plsc.VectorSubcoreMesh(core_axis_name="c", subcore_axis_name="s")

    @functools.partial(
        pl.kernel,
        mesh=mesh,
        out_type=jax.ShapeDtypeStruct((BATCH, SEQ, DIM), jnp.float32),
        compiler_params=pltpu.CompilerParams(use_tc_tiling_on_sc=False),
        scratch_types=[
            pltpu.VMEM((IDX_PER_W,), jnp.int32),
            pltpu.VMEM((SEQ, DIM), jnp.float32),
            pltpu.VMEM((SEQ, DIM), jnp.float32),
            pltpu.SemaphoreType.DMA,
        ],
    )
    def k(x_hbm, tok_hbm, pos_hbm, out_hbm, idx_v, pos_v, buf_v, sem):
        wid = lax.axis_index("s") * NC + lax.axis_index("c")
        pltpu.sync_copy(x_hbm.at[wid], idx_v)
        pltpu.sync_copy(pos_hbm, pos_v)

        def seq_body(i, carry):
            base = i * SEQ
            cp0 = pltpu.async_copy(
                tok_hbm.at[idx_v.at[pl.ds(base, 128)]],
                buf_v.at[pl.ds(0, 128)], sem)
            cp1 = pltpu.async_copy(
                tok_hbm.at[idx_v.at[pl.ds(base + 128, 72)]],
                buf_v.at[pl.ds(128, 72)], sem)
            cp0.wait()
            cp1.wait()

            def add_body(r, c2):
                for rr in range(2):
                    for j in range(DIM // 16):
                        plsc.addupdate(
                            buf_v.at[2 * r + rr, pl.ds(j * 16, 16)],
                            pos_v[2 * r + rr, pl.ds(j * 16, 16)])
                return c2

            lax.fori_loop(0, SEQ // 2, add_body, 0)
            pltpu.sync_copy(buf_v, out_hbm.at[wid * SEQ_PER_W + i])
            return carry

        lax.fori_loop(0, SEQ_PER_W, seq_body, 0)

    return k(x_flat, token_table, pos_table)


def kernel(x, token_table, pos_table):
    x_r = x.astype(jnp.int32).reshape(NW, IDX_PER_W)
    return _sc_embed(x_r, token_table, pos_table)
